# baseline (speedup 1.0000x reference)
import jax
import jax.numpy as jnp
from jax.experimental import pallas as pl
from jax.experimental.pallas import tpu as pltpu
from jax.experimental.pallas import tpu_sc as plsc

_R_BLK = 128
_D_BLK = 128


def kernel(x, times, pos_table, W_time, b_time):
    B, T, D = x.shape
    BT = B * T
    x2 = x.reshape(BT, D)
    t2 = times.reshape(1, BT)
    w2 = W_time.reshape(1, D)
    b2 = b_time.reshape(1, D)

    n_r = BT // _R_BLK
    n_d = D // _D_BLK
    pos_r = T // _R_BLK

    mesh = plsc.VectorSubcoreMesh(core_axis_name="core",
                                  subcore_axis_name="subcore")

    @pl.kernel(out_type=jax.ShapeDtypeStruct((BT, D), x.dtype), mesh=mesh)
    def sc_kernel(x_hbm, t_hbm, w_hbm, b_hbm, pos_hbm, o_hbm):
        def body(x_v, t_v, w_v, b_v, pos_v, o_v):
            w_vecs = [w_v.at[0][pl.ds(j, 16)][...] for j in range(0, _D_BLK, 16)]
            b_vecs = [b_v.at[0][pl.ds(j, 16)][...] for j in range(0, _D_BLK, 16)]

            @pl.loop(0, _R_BLK, step=16)
            def _(g):
                t16 = t_v.at[0][pl.ds(g, 16)][...]
                for rr in range(16):
                    t_r = t16[rr]
                    row = pl.ds(g + rr, 1)
                    for jj, j in enumerate(range(0, _D_BLK, 16)):
                        sl = (row, pl.ds(j, 16))
                        o_v.at[*sl][...] = (
                            x_v.at[*sl][...]
                            + pos_v.at[*sl][...]
                            + (t_r * w_vecs[jj] + b_vecs[jj])
                        )

        pltpu.emit_pipeline(
            body,
            grid=(n_r, n_d),
            in_specs=[
                pl.BlockSpec((_R_BLK, _D_BLK), lambda i, j: (i, j)),
                pl.BlockSpec((1, _R_BLK), lambda i, j: (0, i)),
                pl.BlockSpec((1, _D_BLK), lambda i, j: (0, j)),
                pl.BlockSpec((1, _D_BLK), lambda i, j: (0, j)),
                pl.BlockSpec((_R_BLK, _D_BLK), lambda i, j: (i % pos_r, j)),
            ],
            out_specs=[pl.BlockSpec((_R_BLK, _D_BLK), lambda i, j: (i, j))],
            core_axis_name=("core", "subcore"),
            dimension_semantics=(pltpu.PARALLEL, pltpu.PARALLEL),
        )(x_hbm, t_hbm, w_hbm, b_hbm, pos_hbm, o_hbm)

    out = sc_kernel(x2, t2, w2, b2, pos_table)
    return out.reshape(B, T, D)


# speedup vs baseline: 4.5407x; 4.5407x over previous
import jax
import jax.numpy as jnp
from jax.experimental import pallas as pl


_T_BLK = 2048


def _body(times_ref, w_ref, b_ref, x_ref, pos_ref, o_ref):
    ti = pl.program_id(0)
    tt = times_ref[0, 0, pl.ds(ti * _T_BLK, _T_BLK)]
    w = w_ref[0, :]
    bb = b_ref[0, :]
    time_emb = tt[:, None] * w[None, :] + bb[None, :]
    o_ref[0] = x_ref[0] + pos_ref[...] + time_emb


def kernel(x, times, pos_table, W_time, b_time):
    B, T, D = x.shape
    n_t = T // _T_BLK
    times3 = times.reshape(B, 1, T)
    w2 = W_time.reshape(1, D)
    b2 = b_time.reshape(1, D)

    grid = (n_t, B)
    out = pl.pallas_call(
        _body,
        grid=grid,
        in_specs=[
            pl.BlockSpec((1, 1, T), lambda ti, bi: (bi, 0, 0)),
            pl.BlockSpec((1, D), lambda ti, bi: (0, 0)),
            pl.BlockSpec((1, D), lambda ti, bi: (0, 0)),
            pl.BlockSpec((1, _T_BLK, D), lambda ti, bi: (bi, ti, 0)),
            pl.BlockSpec((_T_BLK, D), lambda ti, bi: (ti, 0)),
        ],
        out_specs=pl.BlockSpec((1, _T_BLK, D), lambda ti, bi: (bi, ti, 0)),
        out_shape=jax.ShapeDtypeStruct((B, T, D), x.dtype),
    )(times3, w2, b2, x, pos_table)
    return out


# speedup vs baseline: 4.6438x; 1.0227x over previous
import jax
import jax.numpy as jnp
from jax.experimental import pallas as pl


_T_BLK = 512


def _body(times_ref, w_ref, b_ref, x_ref, pos_ref, o_ref):
    ti = pl.program_id(0)
    B = times_ref.shape[0]
    tt = times_ref[pl.ds(0, B), 0, pl.ds(ti * _T_BLK, _T_BLK)]
    w = w_ref[0, :]
    bb = b_ref[0, :]
    time_emb = tt[:, :, None] * w[None, None, :] + bb[None, None, :]
    o_ref[...] = x_ref[...] + pos_ref[...][None, :, :] + time_emb


def kernel(x, times, pos_table, W_time, b_time):
    B, T, D = x.shape
    n_t = T // _T_BLK
    times3 = times.reshape(B, 1, T)
    w2 = W_time.reshape(1, D)
    b2 = b_time.reshape(1, D)

    out = pl.pallas_call(
        _body,
        grid=(n_t,),
        in_specs=[
            pl.BlockSpec((B, 1, T), lambda ti: (0, 0, 0)),
            pl.BlockSpec((1, D), lambda ti: (0, 0)),
            pl.BlockSpec((1, D), lambda ti: (0, 0)),
            pl.BlockSpec((B, _T_BLK, D), lambda ti: (0, ti, 0)),
            pl.BlockSpec((_T_BLK, D), lambda ti: (ti, 0)),
        ],
        out_specs=pl.BlockSpec((B, _T_BLK, D), lambda ti: (0, ti, 0)),
        out_shape=jax.ShapeDtypeStruct((B, T, D), x.dtype),
    )(times3, w2, b2, x, pos_table)
    return out
